# baseline (speedup 1.0000x reference)
import jax
import jax.numpy as jnp
from jax import lax
from jax.experimental import pallas as pl
from jax.experimental.pallas import tpu as pltpu
from jax.experimental.pallas import tpu_sc as plsc

N_NODES = 100000
N_EDGES = 3200000
OUT_DIM = 16
CUTOFF = 5.0

NC = 2
NS = 16
NW = NC * NS
E_PER_W = N_EDGES // NW
CHUNK = 800
NCHUNK = E_PER_W // CHUNK
GSUB = 80
NG = CHUNK // GSUB

WIDTH = CUTOFF / (OUT_DIM - 1)
NEG_S = -1.0 / (2.0 * WIDTH * WIDTH)
CENTERS = [k * WIDTH for k in range(OUT_DIM)]


def _rsqrt_nr(d2):
    d2c = jnp.maximum(d2, 1e-30)
    i = plsc.bitcast(d2c, jnp.int32)
    i = 0x5F3759DF - lax.shift_right_logical(i, 1)
    y = plsc.bitcast(i, jnp.float32)
    nh = d2c * -0.5
    for _ in range(3):
        y = y * (1.5 + nh * y * y)
    return y


def _sc_body(px_hbm, py_hbm, pz_hbm, src_hbm, dst_hbm, out_hbm,
             sidx, didx, sx, sy, sz, tx, ty, tz, outv, sem):
    wid = lax.axis_index("s") * NC + lax.axis_index("c")
    ids0 = lax.iota(jnp.int32, 16)

    def chunk_body(i, _):
        base = wid * E_PER_W + i * CHUNK
        pltpu.sync_copy(src_hbm.at[pl.ds(base, CHUNK)], sidx)
        pltpu.sync_copy(dst_hbm.at[pl.ds(base, CHUNK)], didx)
        for j in range(NG):
            sl = pl.ds(j * GSUB, GSUB)
            pltpu.async_copy(px_hbm.at[sidx.at[sl]], sx.at[sl], sem)
            pltpu.async_copy(py_hbm.at[sidx.at[sl]], sy.at[sl], sem)
            pltpu.async_copy(pz_hbm.at[sidx.at[sl]], sz.at[sl], sem)
            pltpu.async_copy(px_hbm.at[didx.at[sl]], tx.at[sl], sem)
            pltpu.async_copy(py_hbm.at[didx.at[sl]], ty.at[sl], sem)
            pltpu.async_copy(pz_hbm.at[didx.at[sl]], tz.at[sl], sem)
        for _buf in range(6):
            pltpu.make_async_copy(px_hbm.at[pl.ds(0, CHUNK)], sx, sem).wait()

        def grp_body(g, _):
            gsl = pl.ds(g * 16, 16)
            dx = sx[gsl] - tx[gsl]
            dy = sy[gsl] - ty[gsl]
            dz = sz[gsl] - tz[gsl]
            d2 = dx * dx + dy * dy + dz * dz
            norm = d2 * _rsqrt_nr(d2)
            ids = ids0 + g * 16
            for k in range(OUT_DIM):
                t = norm - CENTERS[k]
                e = jnp.exp(t * (t * NEG_S))
                plsc.store_scatter(outv, [ids, jnp.full((16,), k, jnp.int32)], e)
            return 0

        lax.fori_loop(0, CHUNK // 16, grp_body, 0)
        pltpu.sync_copy(outv, out_hbm.at[pl.ds(base, CHUNK)])
        return 0

    lax.fori_loop(0, NCHUNK, chunk_body, 0)


@jax.jit
def _sc_rbf(px, py, pz, src, dst):
    mesh = plsc.VectorSubcoreMesh(core_axis_name="c", subcore_axis_name="s")
    return pl.kernel(
        _sc_body,
        out_type=jax.ShapeDtypeStruct((N_EDGES, OUT_DIM), jnp.float32),
        mesh=mesh,
        compiler_params=pltpu.CompilerParams(needs_layout_passes=False),
        scratch_types=[
            pltpu.VMEM((CHUNK,), jnp.int32),
            pltpu.VMEM((CHUNK,), jnp.int32),
            pltpu.VMEM((CHUNK,), jnp.float32),
            pltpu.VMEM((CHUNK,), jnp.float32),
            pltpu.VMEM((CHUNK,), jnp.float32),
            pltpu.VMEM((CHUNK,), jnp.float32),
            pltpu.VMEM((CHUNK,), jnp.float32),
            pltpu.VMEM((CHUNK,), jnp.float32),
            pltpu.VMEM((CHUNK, OUT_DIM), jnp.float32),
            pltpu.SemaphoreType.DMA,
        ],
    )(px, py, pz, src, dst)


def kernel(pos, edge_index):
    return _sc_rbf(pos[:, 0], pos[:, 1], pos[:, 2],
                   edge_index[0], edge_index[1])


# speedup vs baseline: 2.6207x; 2.6207x over previous
import jax
import jax.numpy as jnp
from jax import lax
from jax.experimental import pallas as pl
from jax.experimental.pallas import tpu as pltpu
from jax.experimental.pallas import tpu_sc as plsc

N_NODES = 100000
N_EDGES = 3200000
OUT_DIM = 16
CUTOFF = 5.0

NC = 2
NS = 16
NW = NC * NS
CHUNK = 3200
NCH_TOT = N_EDGES // CHUNK
GSUB = 128
NG = CHUNK // GSUB

WIDTH = CUTOFF / (OUT_DIM - 1)
NEG_S = -1.0 / (2.0 * WIDTH * WIDTH)
CENTERS = [k * WIDTH for k in range(OUT_DIM)]


def _rsqrt_nr(d2):
    d2c = jnp.maximum(d2, 1e-30)
    i = plsc.bitcast(d2c, jnp.int32)
    i = 0x5F3759DF - lax.shift_right_logical(i, 1)
    y = plsc.bitcast(i, jnp.float32)
    nh = d2c * -0.5
    for _ in range(3):
        y = y * (1.5 + nh * y * y)
    return y


def _sc_body(px_hbm, py_hbm, pz_hbm, src_hbm, dst_hbm, out_hbm,
             sidx, didx, sx, sy, sz, tx, ty, tz, outv, sem):
    wid = lax.axis_index("s") * NC + lax.axis_index("c")
    nch = (NCH_TOT - wid + NW - 1) // NW

    def chunk_body(i, _):
        base = (wid + i * NW) * CHUNK
        pltpu.sync_copy(src_hbm.at[pl.ds(base, CHUNK)], sidx)
        pltpu.sync_copy(dst_hbm.at[pl.ds(base, CHUNK)], didx)
        for j in range(NG):
            sl = pl.ds(j * GSUB, GSUB)
            pltpu.async_copy(px_hbm.at[sidx.at[sl]], sx.at[sl], sem)
            pltpu.async_copy(py_hbm.at[sidx.at[sl]], sy.at[sl], sem)
            pltpu.async_copy(pz_hbm.at[sidx.at[sl]], sz.at[sl], sem)
            pltpu.async_copy(px_hbm.at[didx.at[sl]], tx.at[sl], sem)
            pltpu.async_copy(py_hbm.at[didx.at[sl]], ty.at[sl], sem)
            pltpu.async_copy(pz_hbm.at[didx.at[sl]], tz.at[sl], sem)
        for _buf in range(6):
            pltpu.make_async_copy(px_hbm.at[pl.ds(0, CHUNK)], sx, sem).wait()

        def grp_body(g, _):
            gsl = pl.ds(g * 16, 16)
            dx = sx[gsl] - tx[gsl]
            dy = sy[gsl] - ty[gsl]
            dz = sz[gsl] - tz[gsl]
            d2 = dx * dx + dy * dy + dz * dz
            norm = d2 * _rsqrt_nr(d2)
            for k in range(OUT_DIM):
                t = norm - CENTERS[k]
                outv[k, gsl] = jnp.exp(t * (t * NEG_S))
            return 0

        lax.fori_loop(0, CHUNK // 16, grp_body, 0)
        pltpu.sync_copy(outv, out_hbm.at[:, pl.ds(base, CHUNK)])
        return 0

    lax.fori_loop(0, nch, chunk_body, 0)


@jax.jit
def _sc_rbf(px, py, pz, src, dst):
    mesh = plsc.VectorSubcoreMesh(core_axis_name="c", subcore_axis_name="s")
    return pl.kernel(
        _sc_body,
        out_type=jax.ShapeDtypeStruct((OUT_DIM, N_EDGES), jnp.float32),
        mesh=mesh,
        compiler_params=pltpu.CompilerParams(needs_layout_passes=False),
        scratch_types=[
            pltpu.VMEM((CHUNK,), jnp.int32),
            pltpu.VMEM((CHUNK,), jnp.int32),
            pltpu.VMEM((CHUNK,), jnp.float32),
            pltpu.VMEM((CHUNK,), jnp.float32),
            pltpu.VMEM((CHUNK,), jnp.float32),
            pltpu.VMEM((CHUNK,), jnp.float32),
            pltpu.VMEM((CHUNK,), jnp.float32),
            pltpu.VMEM((CHUNK,), jnp.float32),
            pltpu.VMEM((OUT_DIM, CHUNK), jnp.float32),
            pltpu.SemaphoreType.DMA,
        ],
    )(px, py, pz, src, dst)


def kernel(pos, edge_index):
    out_t = _sc_rbf(pos[:, 0], pos[:, 1], pos[:, 2],
                    edge_index[0], edge_index[1])
    return out_t.T
